# baseline (speedup 1.0000x reference)
import functools

import jax
import jax.numpy as jnp
from jax.experimental import pallas as pl
from jax.experimental.pallas import tpu as pltpu

_B = 16


def _mlp_head_body(x1_ref, x2_ref, pt_ref, w1_ref, b1_ref, w2_ref, b2_ref,
                   w3_ref, b3_ref, out_ref):
    x = jnp.concatenate([x1_ref[...], x2_ref[...], pt_ref[...]], axis=-1)
    h = jnp.maximum(jnp.dot(x, w1_ref[...], preferred_element_type=jnp.float32)
                    + b1_ref[...], 0.0)
    h = jnp.maximum(jnp.dot(h, w2_ref[...], preferred_element_type=jnp.float32)
                    + b2_ref[...], 0.0)
    out_ref[...] = (jnp.dot(h, w3_ref[...], preferred_element_type=jnp.float32)
                    + b3_ref[...])


@functools.partial(jax.jit, static_argnames=())
def _mlp_head(x1, x2, point, lin_params):
    (w1, b1), (w2, b2), (w3, b3) = lin_params
    return pl.pallas_call(
        _mlp_head_body,
        out_shape=jax.ShapeDtypeStruct((_B, w3.shape[1]), jnp.float32),
    )(x1, x2, point, w1, b1.reshape(1, -1), w2, b2.reshape(1, -1),
      w3, b3.reshape(1, -1))


def _graph_conv_net(x, edge_index, batch, params, num_graphs):
    src = edge_index[0].astype(jnp.int32)
    dst = edge_index[1].astype(jnp.int32)
    for (W_rel, W_root, b) in params:
        msgs = x[src]
        agg = jnp.zeros((x.shape[0], x.shape[1]), dtype=x.dtype).at[dst].add(msgs)
        x = jax.nn.elu(agg @ W_rel + x @ W_root + b)
    sums = jax.ops.segment_sum(x, batch, num_segments=num_graphs)
    cnt = jax.ops.segment_sum(jnp.ones((x.shape[0], 1), dtype=x.dtype), batch,
                              num_segments=num_graphs)
    return sums / jnp.maximum(cnt, 1.0)


def kernel(graph_x, graph_edge_index, graph_batch, subgraph_x,
           subgraph_edge_index, subgraph_batch, point, g_params, s_params,
           lin_params):
    x1 = _graph_conv_net(graph_x, graph_edge_index, graph_batch, g_params, _B)
    x2 = _graph_conv_net(subgraph_x, subgraph_edge_index, subgraph_batch,
                         s_params, _B)
    return _mlp_head(x1, x2, point, lin_params)


# speedup vs baseline: 1.9842x; 1.9842x over previous
import functools

import jax
import jax.numpy as jnp
from jax import lax
from jax.experimental import pallas as pl
from jax.experimental.pallas import tpu as pltpu
from jax.experimental.pallas import tpu_sc as plsc

_B = 16
_N = 10000
_KB = 128
_AGG_ROWS = 10016
_DUMMY = 10008
_NB = 10
_BN = _N // _NB
_F32 = jnp.float32


def _elu(v):
    return jnp.where(v > 0, v, jnp.exp(jnp.minimum(v, 0.0)) - 1.0)


@functools.cache
def _sc_agg(nb, nc):
    mesh = plsc.VectorSubcoreMesh(core_axis_name="c", subcore_axis_name="s")

    def body(x_flat, src_h, dst_h, zeros_h, out, src_scr, dst_scr, gbuf,
             agg, gsem):
        cid = lax.axis_index("c")
        tid = lax.axis_index("s")
        base = tid * 624

        for c in range(nc):
            pltpu.sync_copy(src_h.at[c, cid, tid], src_scr)
            if c == 0:
                pltpu.sync_copy(dst_h.at[0, cid, tid], dst_scr)
            pltpu.sync_copy(zeros_h, gbuf)
            for off in range(0, 512, 128):
                pltpu.sync_copy(gbuf, agg.at[pl.ds(base + off, 128)])
            pltpu.sync_copy(gbuf.at[pl.ds(0, 112)],
                            agg.at[pl.ds(base + 512, 112)])

            @pl.when(tid == 15)
            def _():
                pltpu.sync_copy(gbuf, agg.at[pl.ds(9872, 128)])

            plsc.subcore_barrier()

            def step(b, carry):
                pltpu.async_copy(x_flat.at[src_scr.at[b]], gbuf, gsem).wait()
                pltpu.sync_copy(gbuf, agg.at[dst_scr.at[b]], add=True)
                return carry

            lax.fori_loop(0, nb, step, 0)
            plsc.subcore_barrier()

            pltpu.sync_copy(agg.at[pl.ds(base, 624)],
                            out.at[cid, c, pl.ds(base, 624)])

            @pl.when(tid == 15)
            def _():
                pltpu.sync_copy(agg.at[pl.ds(9984, 16)],
                                out.at[cid, c, pl.ds(9984, 16)])

            if c < nc - 1:
                plsc.subcore_barrier()

    return pl.kernel(
        body,
        out_type=jax.ShapeDtypeStruct((2, nc, _N, 128), _F32),
        mesh=mesh,
        scratch_types=[
            pltpu.VMEM((nb, _KB), jnp.int32),
            pltpu.VMEM((nb, _KB), jnp.int32),
            pltpu.VMEM((_KB, 128), _F32),
            pltpu.VMEM_SHARED((_AGG_ROWS, 128), _F32),
            pltpu.SemaphoreType.DMA,
        ],
    )


def _conv_body(p_ref, x_ref, wrel_ref, wroot_ref, b_ref, out_ref, acc):
    ci = pl.program_id(2)
    nc_in = pl.num_programs(2)

    @pl.when(ci == 0)
    def _():
        acc[...] = jnp.zeros_like(acc)

    aggb = p_ref[0, 0] + p_ref[1, 0]
    acc[...] += (jnp.dot(aggb, wrel_ref[...], preferred_element_type=_F32)
                 + jnp.dot(x_ref[0], wroot_ref[...],
                           preferred_element_type=_F32))

    @pl.when(ci == nc_in - 1)
    def _():
        out_ref[0] = _elu(acc[...] + b_ref[...])


def _conv_tc(P, X, wrel, wroot, b, nc_in, nc_out):
    return pl.pallas_call(
        _conv_body,
        grid=(_NB, nc_out, nc_in),
        in_specs=[
            pl.BlockSpec((2, 1, _BN, 128), lambda n, co, ci: (0, ci, n, 0)),
            pl.BlockSpec((1, _BN, 128), lambda n, co, ci: (ci, n, 0)),
            pl.BlockSpec((128, 128), lambda n, co, ci: (ci, co)),
            pl.BlockSpec((128, 128), lambda n, co, ci: (ci, co)),
            pl.BlockSpec((1, 128), lambda n, co, ci: (0, co)),
        ],
        out_specs=pl.BlockSpec((1, _BN, 128), lambda n, co, ci: (co, n, 0)),
        out_shape=jax.ShapeDtypeStruct((nc_out, _N, 128), _F32),
        scratch_shapes=[pltpu.VMEM((_BN, 128), _F32)],
    )(P, X, wrel, wroot, b.reshape(1, -1))


def _pre3_body(x_ref, wrel_ref, wroot_ref, b_ref, y_ref, r_ref, accy, accr):
    ci = pl.program_id(2)
    nc_in = pl.num_programs(2)

    @pl.when(ci == 0)
    def _():
        accy[...] = jnp.zeros_like(accy)
        accr[...] = jnp.zeros_like(accr)

    accy[...] += jnp.dot(x_ref[0], wrel_ref[...], preferred_element_type=_F32)
    accr[...] += jnp.dot(x_ref[0], wroot_ref[...],
                         preferred_element_type=_F32)

    @pl.when(ci == nc_in - 1)
    def _():
        y_ref[0] = accy[...]
        r_ref[0] = accr[...] + b_ref[...]


def _pre3_tc(X, wrel, wroot, b, nc_in, nc_out):
    return pl.pallas_call(
        _pre3_body,
        grid=(_NB, nc_out, nc_in),
        in_specs=[
            pl.BlockSpec((1, _BN, 128), lambda n, co, ci: (ci, n, 0)),
            pl.BlockSpec((128, 128), lambda n, co, ci: (ci, co)),
            pl.BlockSpec((128, 128), lambda n, co, ci: (ci, co)),
            pl.BlockSpec((1, 128), lambda n, co, ci: (0, co)),
        ],
        out_specs=[
            pl.BlockSpec((1, _BN, 128), lambda n, co, ci: (co, n, 0)),
            pl.BlockSpec((1, _BN, 128), lambda n, co, ci: (co, n, 0)),
        ],
        out_shape=[
            jax.ShapeDtypeStruct((nc_out, _N, 128), _F32),
            jax.ShapeDtypeStruct((nc_out, _N, 128), _F32),
        ],
        scratch_shapes=[pltpu.VMEM((_BN, 128), _F32),
                        pltpu.VMEM((_BN, 128), _F32)],
    )(X, wrel, wroot, b.reshape(1, -1))


def _pool_body(p_ref, r_ref, batch_ref, sums_ref, cnt_ref, accs, accc):
    co = pl.program_id(0)
    n = pl.program_id(1)

    @pl.when(n == 0)
    def _():
        accs[...] = jnp.zeros_like(accs)
        accc[...] = jnp.zeros_like(accc)

    x3 = _elu(p_ref[0, 0] + p_ref[1, 0] + r_ref[0])
    bt = batch_ref[0]
    seg = lax.broadcasted_iota(jnp.int32, (_B, _BN), 0)
    S = (seg == jnp.broadcast_to(bt, (_B, _BN))).astype(_F32)
    accs[...] += jnp.dot(S, x3, preferred_element_type=_F32)

    @pl.when(co == 0)
    def _():
        accc[...] += jnp.broadcast_to(
            jnp.sum(S, axis=1, keepdims=True), (_B, 128))

    @pl.when(n == _NB - 1)
    def _():
        sums_ref[...] = accs[...]

        @pl.when(co == 0)
        def _():
            cnt_ref[...] = accc[...]


def _pool_tc(P, R, batch3d, nc_out):
    return pl.pallas_call(
        _pool_body,
        grid=(nc_out, _NB),
        in_specs=[
            pl.BlockSpec((2, 1, _BN, 128), lambda co, n: (0, co, n, 0)),
            pl.BlockSpec((1, _BN, 128), lambda co, n: (co, n, 0)),
            pl.BlockSpec((1, 1, _BN), lambda co, n: (n, 0, 0)),
        ],
        out_specs=[
            pl.BlockSpec((_B, 128), lambda co, n: (0, co)),
            pl.BlockSpec((_B, 128), lambda co, n: (0, 0)),
        ],
        out_shape=[
            jax.ShapeDtypeStruct((_B, 128 * nc_out), _F32),
            jax.ShapeDtypeStruct((_B, 128), _F32),
        ],
        scratch_shapes=[pltpu.VMEM((_B, 128), _F32),
                        pltpu.VMEM((_B, 128), _F32)],
    )(P, R, batch3d)


def _head_body(gs_ref, gc_ref, ss_ref, sc_ref, pt_ref, w1_ref, b1_ref,
               w2_ref, b2_ref, w3_ref, b3_ref, out_ref):
    x1 = gs_ref[...] / jnp.maximum(gc_ref[:, 0:1], 1.0)
    x2 = ss_ref[...] / jnp.maximum(sc_ref[:, 0:1], 1.0)
    x = jnp.concatenate([x1, x2, pt_ref[...]], axis=-1)
    h = jnp.maximum(jnp.dot(x, w1_ref[...], preferred_element_type=_F32)
                    + b1_ref[...], 0.0)
    h = jnp.maximum(jnp.dot(h, w2_ref[...], preferred_element_type=_F32)
                    + b2_ref[...], 0.0)
    out_ref[...] = (jnp.dot(h, w3_ref[...], preferred_element_type=_F32)
                    + b3_ref[...])


def _head_tc(gs, gc, ss, sc_, point, lin_params):
    (w1, b1), (w2, b2), (w3, b3) = lin_params
    return pl.pallas_call(
        _head_body,
        out_shape=jax.ShapeDtypeStruct((_B, w3.shape[1]), _F32),
    )(gs, gc, ss, sc_, point, w1, b1.reshape(1, -1), w2, b2.reshape(1, -1),
      w3, b3.reshape(1, -1))


def _prep_edges(edge_index, n_edges):
    src = edge_index[0].astype(jnp.int32)
    dst = edge_index[1].astype(jnp.int32)
    per_tile = -(-n_edges // 32 // _KB) * _KB
    nb = per_tile // _KB
    e_pad = 32 * per_tile
    srcp = jnp.concatenate(
        [src, jnp.zeros((e_pad - n_edges,), jnp.int32)])
    dstp = jnp.concatenate(
        [dst, jnp.full((e_pad - n_edges,), _DUMMY, jnp.int32)])
    dst_h = dstp.reshape(1, 2, 16, nb, _KB)
    src_hs = {}
    for nc in (1, 2, 3):
        offs = (jnp.arange(nc, dtype=jnp.int32) * _N)[:, None]
        src_hs[nc] = (srcp[None, :] + offs).reshape(nc, 2, 16, nb, _KB)
    return src_hs, dst_h, nb


def _conv_net(x0, edge_index, batch, params, n_edges, zeros128):
    src_hs, dst_h, nb = _prep_edges(edge_index, n_edges)
    (wr1, wo1, b1), (wr2, wo2, b2), (wr3, wo3, b3) = params

    X = x0.reshape(1, _N, 128)
    P1 = _sc_agg(nb, 1)(x0, src_hs[1], dst_h, zeros128)
    X2 = _conv_tc(P1, X, wr1, wo1, b1, 1, 2)

    P2 = _sc_agg(nb, 2)(X2.reshape(2 * _N, 128), src_hs[2], dst_h, zeros128)
    X3 = _conv_tc(P2, X2, wr2, wo2, b2, 2, 4)

    Y, R = _pre3_tc(X3, wr3, wo3, b3, 4, 3)
    P3 = _sc_agg(nb, 3)(Y.reshape(3 * _N, 128), src_hs[3], dst_h, zeros128)

    batch3d = batch.astype(jnp.int32).reshape(_NB, 1, _BN)
    return _pool_tc(P3, R, batch3d, 3)


def kernel(graph_x, graph_edge_index, graph_batch, subgraph_x,
           subgraph_edge_index, subgraph_batch, point, g_params, s_params,
           lin_params):
    zeros128 = jnp.zeros((128, 128), _F32)
    gs, gc = _conv_net(graph_x, graph_edge_index, graph_batch, g_params,
                       320000, zeros128)
    ss, sc_ = _conv_net(subgraph_x, subgraph_edge_index, subgraph_batch,
                        s_params, 160000, zeros128)
    return _head_tc(gs, gc, ss, sc_, point, lin_params)
